# Initial kernel scaffold; baseline (speedup 1.0000x reference)
import jax
import jax.numpy as jnp
from jax.experimental import pallas as pl


def kernel(node_features, edge_features, edge_index, node_tiers, w_node, w_edge, Wq, Wk, Wv, Wo, W1, b1, W2, b2):
    raise NotImplementedError("write your pallas kernel here")


# speedup vs baseline: 2.9989x; 2.9989x over previous
import functools
import math

import jax
import jax.numpy as jnp
from jax.experimental import pallas as pl

N = 10000
E = 320000
D_NODE = 128
D_EDGE = 16
N_CLASSES = 8
N_HEADS = 4
DH = D_EDGE // N_HEADS
EB = 4000


def _node_proj_k(nf_ref, wn_ref, wq_ref, ns_ref, q_ref):
    nf = nf_ref[...]
    ns_ref[...] = nf @ wn_ref[...]
    q_ref[...] = nf @ wq_ref[...]


def _edge_score_k(ef_ref, nss_ref, nsd_ref, we_ref, es_ref):
    es_ref[...] = ef_ref[...] @ we_ref[...] + 0.5 * (nss_ref[...] + nsd_ref[...])


def _attn_stage1_k(ef_ref, mf_ref, qd_ref, wk_ref, wv_ref, efm_ref, v_ref, lg_ref):
    efm = ef_ref[...] * mf_ref[...]
    efm_ref[...] = efm
    kk = efm @ wk_ref[...]
    v_ref[...] = efm @ wv_ref[...]
    qd = qd_ref[...]
    scale = 1.0 / math.sqrt(float(DH))
    cols = []
    for h in range(N_HEADS):
        s = slice(h * DH, (h + 1) * DH)
        cols.append(jnp.sum(qd[:, s] * kk[:, s], axis=1, keepdims=True) * scale)
    lg_ref[...] = jnp.concatenate(cols, axis=1)


def _exp_weight_k(lg_ref, md_ref, v_ref, ex_ref, exv_ref):
    ex = jnp.exp(lg_ref[...] - md_ref[...])
    ex_ref[...] = ex
    v = v_ref[...]
    outs = []
    for h in range(N_HEADS):
        s = slice(h * DH, (h + 1) * DH)
        outs.append(ex[:, h:h + 1] * v[:, s])
    exv_ref[...] = jnp.concatenate(outs, axis=1)


def _node_agg_k(den_ref, sv_ref, na_ref):
    den = den_ref[...]
    sv = sv_ref[...]
    outs = []
    for h in range(N_HEADS):
        s = slice(h * DH, (h + 1) * DH)
        outs.append(sv[:, s] / (den[:, h:h + 1] + 1e-9))
    na_ref[...] = jnp.concatenate(outs, axis=1)


def _head_k(nad_ref, efm_ref, wo_ref, w1_ref, b1_ref, w2_ref, b2_ref, o_ref):
    edge_feats = nad_ref[...] @ wo_ref[...] + efm_ref[...]
    hh = jax.nn.gelu(edge_feats @ w1_ref[...] + b1_ref[...])
    o_ref[...] = hh @ w2_ref[...] + b2_ref[...]


def _eblk(width):
    return pl.BlockSpec((EB, width), lambda i: (i, 0))


def _rep(shape):
    return pl.BlockSpec(shape, lambda i: (0, 0))


def _topk_mask_bool(scores, k):
    _, idx = jax.lax.top_k(scores, k)
    return jnp.zeros((scores.shape[0],), dtype=bool).at[idx].set(True)


@jax.jit
def kernel(node_features, edge_features, edge_index, node_tiers,
           w_node, w_edge, Wq, Wk, Wv, Wo, W1, b1, W2, b2):
    del node_tiers
    src = edge_index[0]
    dst = edge_index[1]
    grid = (E // EB,)

    ns, q = pl.pallas_call(
        _node_proj_k,
        grid=(1,),
        in_specs=[_rep((N, D_NODE)), _rep((D_NODE, 1)), _rep((D_NODE, D_EDGE))],
        out_specs=[_rep((N, 1)), _rep((N, D_EDGE))],
        out_shape=[jax.ShapeDtypeStruct((N, 1), jnp.float32),
                   jax.ShapeDtypeStruct((N, D_EDGE), jnp.float32)],
    )(node_features, w_node.reshape(D_NODE, 1), Wq)
    ns1 = ns[:, 0]

    nss = jnp.take(ns1, src)[:, None]
    nsd = jnp.take(ns1, dst)[:, None]

    es = pl.pallas_call(
        _edge_score_k,
        grid=grid,
        in_specs=[_eblk(D_EDGE), _eblk(1), _eblk(1), _rep((D_EDGE, 1))],
        out_specs=_eblk(1),
        out_shape=jax.ShapeDtypeStruct((E, 1), jnp.float32),
    )(edge_features, nss, nsd, w_edge.reshape(D_EDGE, 1))[:, 0]

    node_mask = _topk_mask_bool(ns1, int(N * 0.5))
    edge_mask = (_topk_mask_bool(es, int(E * 0.5))
                 & jnp.take(node_mask, src) & jnp.take(node_mask, dst))
    maskf = edge_mask.astype(jnp.float32)[:, None]

    qd = jnp.take(q, dst, axis=0)

    efm, v, logits = pl.pallas_call(
        _attn_stage1_k,
        grid=grid,
        in_specs=[_eblk(D_EDGE), _eblk(1), _eblk(D_EDGE),
                  _rep((D_EDGE, D_EDGE)), _rep((D_EDGE, D_EDGE))],
        out_specs=[_eblk(D_EDGE), _eblk(D_EDGE), _eblk(N_HEADS)],
        out_shape=[jax.ShapeDtypeStruct((E, D_EDGE), jnp.float32),
                   jax.ShapeDtypeStruct((E, D_EDGE), jnp.float32),
                   jax.ShapeDtypeStruct((E, N_HEADS), jnp.float32)],
    )(edge_features, maskf, qd, Wk, Wv)

    m = jax.ops.segment_max(logits, dst, num_segments=N)
    m = jnp.where(jnp.isfinite(m), m, 0.0)
    md = jnp.take(m, dst, axis=0)

    ex, exv = pl.pallas_call(
        _exp_weight_k,
        grid=grid,
        in_specs=[_eblk(N_HEADS), _eblk(N_HEADS), _eblk(D_EDGE)],
        out_specs=[_eblk(N_HEADS), _eblk(D_EDGE)],
        out_shape=[jax.ShapeDtypeStruct((E, N_HEADS), jnp.float32),
                   jax.ShapeDtypeStruct((E, D_EDGE), jnp.float32)],
    )(logits, md, v)

    seg = jax.ops.segment_sum(jnp.concatenate([ex, exv], axis=1), dst,
                              num_segments=N)
    denom, sumv = seg[:, :N_HEADS], seg[:, N_HEADS:]

    node_agg = pl.pallas_call(
        _node_agg_k,
        grid=(1,),
        in_specs=[_rep((N, N_HEADS)), _rep((N, D_EDGE))],
        out_specs=_rep((N, D_EDGE)),
        out_shape=jax.ShapeDtypeStruct((N, D_EDGE), jnp.float32),
    )(denom, sumv)

    nad = jnp.take(node_agg, dst, axis=0)

    out = pl.pallas_call(
        _head_k,
        grid=grid,
        in_specs=[_eblk(D_EDGE), _eblk(D_EDGE), _rep((D_EDGE, D_EDGE)),
                  _rep((D_EDGE, D_EDGE)), _rep((1, D_EDGE)),
                  _rep((D_EDGE, N_CLASSES)), _rep((1, N_CLASSES))],
        out_specs=_eblk(N_CLASSES),
        out_shape=jax.ShapeDtypeStruct((E, N_CLASSES), jnp.float32),
    )(nad, efm, Wo, W1, b1.reshape(1, D_EDGE), W2, b2.reshape(1, N_CLASSES))

    return out


# speedup vs baseline: 3.0236x; 1.0082x over previous
import functools
import math

import jax
import jax.numpy as jnp
from jax.experimental import pallas as pl

N = 10000
E = 320000
D_NODE = 128
D_EDGE = 16
N_CLASSES = 8
N_HEADS = 4
DH = D_EDGE // N_HEADS
EB = 4000


def _node_proj_k(nf_ref, wn_ref, wq_ref, ns_ref, q_ref):
    nf = nf_ref[...]
    ns_ref[...] = nf @ wn_ref[...]
    q_ref[...] = nf @ wq_ref[...]


def _edge_score_k(ef_ref, nss_ref, nsd_ref, we_ref, es_ref):
    es_ref[...] = ef_ref[...] @ we_ref[...] + 0.5 * (nss_ref[...] + nsd_ref[...])


def _attn_stage1_k(ef_ref, mf_ref, qd_ref, wk_ref, wv_ref, efm_ref, v_ref, lg_ref):
    efm = ef_ref[...] * mf_ref[...]
    efm_ref[...] = efm
    kk = efm @ wk_ref[...]
    v_ref[...] = efm @ wv_ref[...]
    qd = qd_ref[...]
    scale = 1.0 / math.sqrt(float(DH))
    cols = []
    for h in range(N_HEADS):
        s = slice(h * DH, (h + 1) * DH)
        cols.append(jnp.sum(qd[:, s] * kk[:, s], axis=1, keepdims=True) * scale)
    lg_ref[...] = jnp.concatenate(cols, axis=1)


def _exp_weight_k(lg_ref, md_ref, v_ref, o_ref):
    ex = jnp.exp(lg_ref[...] - md_ref[...])
    v = v_ref[...]
    outs = [ex]
    for h in range(N_HEADS):
        s = slice(h * DH, (h + 1) * DH)
        outs.append(ex[:, h:h + 1] * v[:, s])
    o_ref[...] = jnp.concatenate(outs, axis=1)


def _node_agg_k(den_ref, sv_ref, na_ref):
    den = den_ref[...]
    sv = sv_ref[...]
    outs = []
    for h in range(N_HEADS):
        s = slice(h * DH, (h + 1) * DH)
        outs.append(sv[:, s] / (den[:, h:h + 1] + 1e-9))
    na_ref[...] = jnp.concatenate(outs, axis=1)


def _head_k(nad_ref, efm_ref, wo_ref, w1_ref, b1_ref, w2_ref, b2_ref, o_ref):
    edge_feats = nad_ref[...] @ wo_ref[...] + efm_ref[...]
    hh = jax.nn.gelu(edge_feats @ w1_ref[...] + b1_ref[...])
    o_ref[...] = hh @ w2_ref[...] + b2_ref[...]


def _eblk(width):
    return pl.BlockSpec((EB, width), lambda i: (i, 0))


def _rep(shape):
    return pl.BlockSpec(shape, lambda i: (0, 0))


def _topk_mask_bool(scores, k):
    _, idx = jax.lax.top_k(scores, k)
    return jnp.zeros((scores.shape[0],), dtype=bool).at[idx].set(True)


@jax.jit
def kernel(node_features, edge_features, edge_index, node_tiers,
           w_node, w_edge, Wq, Wk, Wv, Wo, W1, b1, W2, b2):
    del node_tiers
    src = edge_index[0]
    dst = edge_index[1]
    grid = (E // EB,)

    ns, q = pl.pallas_call(
        _node_proj_k,
        grid=(1,),
        in_specs=[_rep((N, D_NODE)), _rep((D_NODE, 1)), _rep((D_NODE, D_EDGE))],
        out_specs=[_rep((N, 1)), _rep((N, D_EDGE))],
        out_shape=[jax.ShapeDtypeStruct((N, 1), jnp.float32),
                   jax.ShapeDtypeStruct((N, D_EDGE), jnp.float32)],
    )(node_features, w_node.reshape(D_NODE, 1), Wq)
    ns1 = ns[:, 0]

    nss = jnp.take(ns1, src)[:, None]
    nsd = jnp.take(ns1, dst)[:, None]

    es = pl.pallas_call(
        _edge_score_k,
        grid=grid,
        in_specs=[_eblk(D_EDGE), _eblk(1), _eblk(1), _rep((D_EDGE, 1))],
        out_specs=_eblk(1),
        out_shape=jax.ShapeDtypeStruct((E, 1), jnp.float32),
    )(edge_features, nss, nsd, w_edge.reshape(D_EDGE, 1))[:, 0]

    node_mask = _topk_mask_bool(ns1, int(N * 0.5))
    edge_mask = (_topk_mask_bool(es, int(E * 0.5))
                 & jnp.take(node_mask, src) & jnp.take(node_mask, dst))
    maskf = edge_mask.astype(jnp.float32)[:, None]

    qd = jnp.take(q, dst, axis=0)

    efm, v, logits = pl.pallas_call(
        _attn_stage1_k,
        grid=grid,
        in_specs=[_eblk(D_EDGE), _eblk(1), _eblk(D_EDGE),
                  _rep((D_EDGE, D_EDGE)), _rep((D_EDGE, D_EDGE))],
        out_specs=[_eblk(D_EDGE), _eblk(D_EDGE), _eblk(N_HEADS)],
        out_shape=[jax.ShapeDtypeStruct((E, D_EDGE), jnp.float32),
                   jax.ShapeDtypeStruct((E, D_EDGE), jnp.float32),
                   jax.ShapeDtypeStruct((E, N_HEADS), jnp.float32)],
    )(edge_features, maskf, qd, Wk, Wv)

    m = jax.ops.segment_max(logits, dst, num_segments=N)
    m = jnp.where(jnp.isfinite(m), m, 0.0)
    md = jnp.take(m, dst, axis=0)

    exw = pl.pallas_call(
        _exp_weight_k,
        grid=grid,
        in_specs=[_eblk(N_HEADS), _eblk(N_HEADS), _eblk(D_EDGE)],
        out_specs=_eblk(N_HEADS + D_EDGE),
        out_shape=jax.ShapeDtypeStruct((E, N_HEADS + D_EDGE), jnp.float32),
    )(logits, md, v)

    seg = jax.ops.segment_sum(exw, dst, num_segments=N)
    denom, sumv = seg[:, :N_HEADS], seg[:, N_HEADS:]

    node_agg = pl.pallas_call(
        _node_agg_k,
        grid=(1,),
        in_specs=[_rep((N, N_HEADS)), _rep((N, D_EDGE))],
        out_specs=_rep((N, D_EDGE)),
        out_shape=jax.ShapeDtypeStruct((N, D_EDGE), jnp.float32),
    )(denom, sumv)

    nad = jnp.take(node_agg, dst, axis=0)

    out = pl.pallas_call(
        _head_k,
        grid=grid,
        in_specs=[_eblk(D_EDGE), _eblk(D_EDGE), _rep((D_EDGE, D_EDGE)),
                  _rep((D_EDGE, D_EDGE)), _rep((1, D_EDGE)),
                  _rep((D_EDGE, N_CLASSES)), _rep((1, N_CLASSES))],
        out_specs=_eblk(N_CLASSES),
        out_shape=jax.ShapeDtypeStruct((E, N_CLASSES), jnp.float32),
    )(nad, efm, Wo, W1, b1.reshape(1, D_EDGE), W2, b2.reshape(1, N_CLASSES))

    return out
